# Initial kernel scaffold; baseline (speedup 1.0000x reference)
import jax
import jax.numpy as jnp
from jax.experimental import pallas as pl
from jax.experimental.pallas import tpu as pltpu

_N_CHANNELS = 4096
_N_BINS = 8192
_ROW_BLOCK = 128
_N_ROW_BLOCKS = _N_CHANNELS // _ROW_BLOCK


def _mv_body(params_ref, energies_ref, m_ref, out_ref, flux_ref):
    i = pl.program_id(0)

    @pl.when(i == 0)
    def _():
        alpha = params_ref[0]
        norm = params_ref[1]
        oma = 1.0 - alpha
        e_low = energies_ref[0, :]
        e_high = energies_ref[1, :]
        flux = norm * (jnp.exp(oma * jnp.log(e_high))
                       - jnp.exp(oma * jnp.log(e_low))) / oma
        flux_ref[0, :] = flux

    m = m_ref[...]
    acc = jnp.sum(m * flux_ref[0, :][None, :], axis=1)
    out_ref[0, :] = jnp.maximum(acc, 1e-6)


def kernel(parameters, energies, transfer_matrix):
    out = pl.pallas_call(
        _mv_body,
        grid=(_N_ROW_BLOCKS,),
        in_specs=[
            pl.BlockSpec(memory_space=pltpu.SMEM),
            pl.BlockSpec((2, _N_BINS), lambda i: (0, 0)),
            pl.BlockSpec((_ROW_BLOCK, _N_BINS), lambda i: (i, 0)),
        ],
        out_specs=pl.BlockSpec((1, _ROW_BLOCK), lambda i: (i, 0)),
        out_shape=jax.ShapeDtypeStruct((_N_ROW_BLOCKS, _ROW_BLOCK), jnp.float32),
        scratch_shapes=[pltpu.VMEM((1, _N_BINS), jnp.float32)],
    )(parameters, energies, transfer_matrix)
    return out.reshape(_N_CHANNELS)


# speedup vs baseline: 1.0089x; 1.0089x over previous
import jax
import jax.numpy as jnp
from jax.experimental import pallas as pl
from jax.experimental.pallas import tpu as pltpu

_N_CHANNELS = 4096
_N_BINS = 8192
_ROW_BLOCK = 128
_N_ROW_BLOCKS = _N_CHANNELS // _ROW_BLOCK


def _mv_body(params_ref, energies_ref, m_ref, out_ref, flux_ref):
    i = pl.program_id(0)

    @pl.when(i == 0)
    def _():
        alpha = params_ref[0]
        norm = params_ref[1]
        oma = 1.0 - alpha
        e_low = energies_ref[0, :]
        e_high = energies_ref[1, :]
        flux = norm * (jnp.exp(oma * jnp.log(e_high))
                       - jnp.exp(oma * jnp.log(e_low))) / oma
        flux_ref[0, :] = flux

    m = m_ref[...]
    acc = jnp.sum(m * flux_ref[0, :][None, :], axis=1)
    out_ref[0, 0, :] = jnp.maximum(acc, 1e-6)


def kernel(parameters, energies, transfer_matrix):
    out = pl.pallas_call(
        _mv_body,
        grid=(_N_ROW_BLOCKS,),
        in_specs=[
            pl.BlockSpec(memory_space=pltpu.SMEM),
            pl.BlockSpec((2, _N_BINS), lambda i: (0, 0)),
            pl.BlockSpec((_ROW_BLOCK, _N_BINS), lambda i: (i, 0)),
        ],
        out_specs=pl.BlockSpec((1, 1, _ROW_BLOCK), lambda i: (i, 0, 0)),
        out_shape=jax.ShapeDtypeStruct((_N_ROW_BLOCKS, 1, _ROW_BLOCK), jnp.float32),
        scratch_shapes=[pltpu.VMEM((1, _N_BINS), jnp.float32)],
    )(parameters, energies, transfer_matrix)
    return out.reshape(_N_CHANNELS)
